# Initial kernel scaffold; baseline (speedup 1.0000x reference)
import jax
import jax.numpy as jnp
from jax.experimental import pallas as pl


def kernel(species_ids, table):
    raise NotImplementedError("write your pallas kernel here")


# speedup vs baseline: 5.9389x; 5.9389x over previous
import functools

import jax
import jax.numpy as jnp
from jax import lax
from jax.experimental import pallas as pl
from jax.experimental.pallas import tpu as pltpu
from jax.experimental.pallas import tpu_sc as plsc

_NUM_SPECIES = 100000
_D = 32
_B = 16384 * 20
_NC = 2
_NS = 16
_NW = _NC * _NS
_BPW = _B // _NW
_CHUNK = 128
_NG = _BPW // _CHUNK

_mesh = plsc.VectorSubcoreMesh(
    core_axis_name="c", subcore_axis_name="s",
    num_cores=_NC, num_subcores=_NS)


@functools.partial(
    pl.kernel,
    out_type=jax.ShapeDtypeStruct((_B, _D), jnp.float32),
    mesh=_mesh,
    compiler_params=pltpu.CompilerParams(use_tc_tiling_on_sc=False),
    scratch_types=[
        pltpu.VMEM((_NG, _CHUNK), jnp.int32),
        pltpu.VMEM((_CHUNK, _D), jnp.float32),
        pltpu.VMEM((_CHUNK, _D), jnp.float32),
        pltpu.SemaphoreType.DMA,
        pltpu.SemaphoreType.DMA,
    ],
)
def _gather_kernel(idx_hbm, table_hbm, out_hbm, idx_v, rows0, rows1,
                   sem0, sem1):
    wid = lax.axis_index("s") * _NC + lax.axis_index("c")
    base = wid * _BPW
    pltpu.sync_copy(idx_hbm.at[wid], idx_v)

    bufs = ((rows0, sem0), (rows1, sem1))

    pltpu.async_copy(table_hbm.at[idx_v.at[0]], rows0, sem0)

    @pl.loop(0, _NG - 1)
    def _body(j):
        cur = j % 2
        nxt = (j + 1) % 2

        @pl.when(cur == 0)
        def _():
            pltpu.async_copy(table_hbm.at[idx_v.at[j + 1]], rows1, sem1)
            pltpu.make_async_copy(table_hbm.at[idx_v.at[j]], rows0,
                                  sem0).wait()
            pltpu.sync_copy(rows0, out_hbm.at[pl.ds(base + j * _CHUNK,
                                                    _CHUNK)])

        @pl.when(cur == 1)
        def _():
            pltpu.async_copy(table_hbm.at[idx_v.at[j + 1]], rows0, sem0)
            pltpu.make_async_copy(table_hbm.at[idx_v.at[j]], rows1,
                                  sem1).wait()
            pltpu.sync_copy(rows1, out_hbm.at[pl.ds(base + j * _CHUNK,
                                                    _CHUNK)])

    last = _NG - 1
    lbuf, lsem = bufs[last % 2]
    pltpu.make_async_copy(table_hbm.at[idx_v.at[last]], lbuf, lsem).wait()
    pltpu.sync_copy(lbuf, out_hbm.at[pl.ds(base + last * _CHUNK, _CHUNK)])


def kernel(species_ids, table):
    idx = species_ids.reshape(_NW, _NG, _CHUNK).astype(jnp.int32)
    out = _gather_kernel(idx, table)
    return out.reshape(species_ids.shape[0], species_ids.shape[1], _D)


# speedup vs baseline: 6.3091x; 1.0623x over previous
import functools

import jax
import jax.numpy as jnp
from jax import lax
from jax.experimental import pallas as pl
from jax.experimental.pallas import tpu as pltpu
from jax.experimental.pallas import tpu_sc as plsc

_BATCH = 16384
_NP = 20
_D = 32
_B = _BATCH * _NP
_NC = 2
_NS = 16
_NW = _NC * _NS
_BPW = _B // _NW
_GROW = 128
_GPC = 5
_CROW = _GROW * _GPC
_CB = _CROW // _NP
_NCHUNK = _BPW // _CROW
_BPWB = _BPW // _NP

_mesh = plsc.VectorSubcoreMesh(
    core_axis_name="c", subcore_axis_name="s",
    num_cores=_NC, num_subcores=_NS)


@functools.partial(
    pl.kernel,
    out_type=jax.ShapeDtypeStruct((_BATCH, _NP, _D), jnp.float32),
    mesh=_mesh,
    compiler_params=pltpu.CompilerParams(use_tc_tiling_on_sc=False),
    scratch_types=[
        pltpu.VMEM((_BPW,), jnp.int32),
        pltpu.VMEM((_CROW, _D), jnp.float32),
        pltpu.VMEM((_CROW, _D), jnp.float32),
        pltpu.SemaphoreType.DMA,
        pltpu.SemaphoreType.DMA,
        pltpu.SemaphoreType.DMA,
        pltpu.SemaphoreType.DMA,
    ],
)
def _gather_kernel(idx_hbm, table_hbm, out_hbm, idx_v, rows0, rows1,
                   gsem0, gsem1, fsem0, fsem1):
    wid = lax.axis_index("s") * _NC + lax.axis_index("c")
    bbase = wid * _BPWB
    pltpu.sync_copy(idx_hbm.at[pl.ds(wid * _BPW, _BPW)], idx_v)

    def fire_gather(k, rows, sem):
        for g in range(_GPC):
            pltpu.async_copy(
                table_hbm.at[idx_v.at[pl.ds(k * _CROW + g * _GROW, _GROW)]],
                rows.at[pl.ds(g * _GROW, _GROW)], sem)

    def drain_gather(k, rows, sem):
        for g in range(_GPC):
            pltpu.make_async_copy(
                table_hbm.at[idx_v.at[pl.ds(k * _CROW + g * _GROW, _GROW)]],
                rows.at[pl.ds(g * _GROW, _GROW)], sem).wait()

    def fire_flush(k, rows, sem):
        for e in range(_CB):
            pltpu.async_copy(rows.at[pl.ds(e * _NP, _NP)],
                             out_hbm.at[bbase + k * _CB + e], sem)

    def drain_flush(k, rows, sem):
        for e in range(_CB):
            pltpu.make_async_copy(rows.at[pl.ds(e * _NP, _NP)],
                                  out_hbm.at[bbase + k * _CB + e],
                                  sem).wait()

    fire_gather(0, rows0, gsem0)

    @pl.loop(0, _NCHUNK, step=2)
    def _body(k):
        fire_gather(k + 1, rows1, gsem1)
        drain_gather(k, rows0, gsem0)
        fire_flush(k, rows0, fsem0)
        drain_gather(k + 1, rows1, gsem1)
        fire_flush(k + 1, rows1, fsem1)
        drain_flush(k, rows0, fsem0)

        @pl.when(k + 2 < _NCHUNK)
        def _():
            fire_gather(k + 2, rows0, gsem0)

        drain_flush(k + 1, rows1, fsem1)


def kernel(species_ids, table):
    return _gather_kernel(species_ids.reshape(-1).astype(jnp.int32), table)
